# baseline (speedup 1.0000x reference)
import functools

import jax
import jax.numpy as jnp
from jax import lax
from jax.experimental import pallas as pl
from jax.experimental.pallas import tpu as pltpu

DIM = 768
NUM_HEADS = 12
HEAD_DIM = 64
HGRID = 24
N = HGRID * HGRID
KEEP = int(N * 0.7)
MM = int(0.7 * KEEP)
SCALE = HEAD_DIM ** -0.5
NSRC = (KEEP + 1) // 2
NDST = KEEP // 2
RR = min(KEEP - MM, KEEP // 2)
NUNM = NSRC - RR
TOPK = MM // 8
B = 32

_F32 = jnp.float32
_BF16 = jnp.bfloat16
_HIGH = lax.Precision.HIGHEST


def _bdot(a, b, dims):
    return lax.dot_general(a.astype(_BF16), b.astype(_BF16), (dims, ((), ())),
                           preferred_element_type=_F32)


def _fdot(a, b, dims):
    return lax.dot_general(a, b, (dims, ((), ())),
                           preferred_element_type=_F32, precision=_HIGH)


def _iota(shape, dim):
    return lax.broadcasted_iota(jnp.int32, shape, dim).astype(_F32)


def _body(x_ref, probs_ref, metric_ref, wqkv_ref, wproj_ref, bproj_ref, o_ref):
    xt = x_ref[0]
    neg_inf = _F32(-jnp.inf)

    i576 = (_iota((N, N), 0) == _iota((N, N), 1)).astype(_F32)
    i202 = (_iota((NSRC, NSRC), 0) == _iota((NSRC, NSRC), 1)).astype(_F32)

    probs_row = probs_ref[0]
    probs_col = _fdot(i576, probs_row, ((1,), (1,)))

    pr_m = jnp.broadcast_to(probs_row, (N, N))
    pr_n = jnp.broadcast_to(probs_col, (N, N))
    lane = _iota((N, N), 1)
    subl = _iota((N, N), 0)
    cmp = (pr_m > pr_n) | ((pr_m == pr_n) & (lane < subl))
    rank_col = jnp.sum(cmp.astype(_F32), axis=1, keepdims=True)
    rank_row = _fdot(rank_col, i576, ((0,), (0,)))

    rr_a = jnp.broadcast_to(rank_row, (NSRC, N))
    p_a = (rr_a == 2.0 * _iota((NSRC, N), 0)).astype(_F32)
    rr_b = jnp.broadcast_to(rank_row, (NDST, N))
    p_b = (rr_b == 2.0 * _iota((NDST, N), 0) + 1.0).astype(_F32)

    metric = metric_ref[0]
    src = _fdot(p_a, xt, ((1,), (0,)))
    dst = _fdot(p_b, xt, ((1,), (0,)))
    a_m = _bdot(p_a, metric, ((1,), (0,)))
    b_m = _bdot(p_b, metric, ((1,), (0,)))

    scores = _bdot(a_m, b_m, ((1,), (1,)))
    msim_col = jnp.max(scores, axis=1, keepdims=True)
    jlane = _iota((NSRC, NDST), 1)
    dstf_col = jnp.min(jnp.where(scores == msim_col, jlane, _F32(NDST)),
                       axis=1, keepdims=True)
    msim_row = _fdot(msim_col, i202, ((0,), (0,)))
    ms_m = jnp.broadcast_to(msim_row, (NSRC, NSRC))
    ms_n = jnp.broadcast_to(msim_col, (NSRC, NSRC))
    lane2 = _iota((NSRC, NSRC), 1)
    subl2 = _iota((NSRC, NSRC), 0)
    cmp2 = (ms_m > ms_n) | ((ms_m == ms_n) & (lane2 < subl2))
    rank2_col = jnp.sum(cmp2.astype(_F32), axis=1, keepdims=True)
    merged_f = (rank2_col < _F32(RR)).astype(_F32)

    s_oh = merged_f * (dstf_col == jlane).astype(_F32)
    sums = _fdot(s_oh, src, ((0,), (0,)))
    counts_col = _fdot(s_oh, jnp.ones((NSRC, 1), _F32), ((0,), (0,)))
    dstm = (dst + sums) / (1.0 + counts_col)

    rank2_row = _fdot(rank2_col, i202, ((0,), (0,)))
    u_oh = (jnp.broadcast_to(rank2_row, (NUNM, NSRC))
            == _F32(RR) + _iota((NUNM, NSRC), 0)).astype(_F32)
    srcu = _bdot(u_oh, src, ((1,), (0,)))

    c1 = (_iota((MM, NUNM), 0) == _iota((MM, NUNM), 1)).astype(_F32)
    c2 = (_iota((MM, NDST), 0) == _iota((MM, NDST), 1) + _F32(NUNM)).astype(_F32)
    xm = _bdot(c1, srcu, ((1,), (0,))) + _fdot(c2, dstm, ((1,), (0,)))

    qkv = _bdot(xm, wqkv_ref[...], ((1,), (1,)))

    outs = []
    for h in range(NUM_HEADS):
        q = qkv[:, h * HEAD_DIM:(h + 1) * HEAD_DIM]
        kk = qkv[:, DIM + h * HEAD_DIM:DIM + (h + 1) * HEAD_DIM]
        v = qkv[:, 2 * DIM + h * HEAD_DIM:2 * DIM + (h + 1) * HEAD_DIM]
        att = _bdot(q, kk, ((1,), (1,))) * _F32(SCALE)
        rmx = jnp.max(att, axis=1, keepdims=True)
        rmn = jnp.min(att, axis=1, keepdims=True)
        ones282 = jnp.ones((MM, 1), _F32)

        def bis_body(_, lh):
            lo, hi = lh
            mid = 0.5 * (lo + hi)
            maskm = jnp.where(att >= mid, 1.0, 0.0)
            cnt = _bdot(maskm, ones282, ((1,), (0,)))
            geq = cnt >= _F32(TOPK)
            return jnp.where(geq, mid, lo), jnp.where(geq, hi, mid)

        lo, _ = lax.fori_loop(0, 22, bis_body, (rmn, rmx))
        maskf = jnp.where(att >= lo, 1.0, 0.0)
        e = jnp.exp(att - rmx) * maskf
        p = e / jnp.sum(e, axis=1, keepdims=True)
        outs.append(_bdot(p, v, ((1,), (0,))))

    jl282 = _iota((NSRC, MM), 1)
    dstf_b = jnp.broadcast_to(dstf_col, (NSRC, MM))
    rank2_b = jnp.broadcast_to(rank2_col, (NSRC, MM))
    merged_b = jnp.broadcast_to(merged_f, (NSRC, MM))
    e_oh = (merged_b * (jl282 == _F32(NUNM) + dstf_b).astype(_F32)
            + (1.0 - merged_b) * (jl282 == rank2_b - _F32(RR)).astype(_F32))
    x2 = (_iota((KEEP, NSRC), 0) == 2.0 * _iota((KEEP, NSRC), 1)).astype(_F32)
    pp = _iota((KEEP, MM), 0)
    jj = _iota((KEEP, MM), 1)
    podd = (pp - 2.0 * jnp.floor(pp * 0.5)) == 1.0
    m_odd = (podd & (2.0 * jj == pp + _F32(160.0) + 1.0)).astype(_F32)
    mx_oh = _bdot(x2, e_oh, ((1,), (0,))) + m_odd
    q_oh = (jnp.broadcast_to(rank_col, (N, KEEP))
            == _iota((N, KEEP), 1)).astype(_F32)
    g_oh = _bdot(q_oh, mx_oh, ((1,), (0,)))

    xp = xt * (rank_col >= _F32(KEEP)).astype(_F32)
    xo = _bdot(xp, wproj_ref[...], ((1,), (1,)))
    for h in range(NUM_HEADS):
        yh = _bdot(g_oh, outs[h], ((1,), (0,)))
        wslice = wproj_ref[:, h * HEAD_DIM:(h + 1) * HEAD_DIM]
        xo = xo + _bdot(yh, wslice, ((1,), (1,)))
    xo = xo + bproj_ref[...]
    o_ref[0] = xo


@jax.jit
def kernel(x, W_g, W_k, W_qkv, W_proj, b_proj):
    xr = x.reshape(B, N, DIM)
    gt = (xr.mean(axis=1) @ W_g.T)[:, None, :]
    k = xr @ W_k.T
    gs = jnp.einsum('bqd,bnd->bqn', gt, k) * SCALE
    probs = jax.nn.softmax(gs[:, 0, :], axis=-1)
    metric = k / jnp.linalg.norm(k, axis=-1, keepdims=True)
    out = pl.pallas_call(
        _body,
        grid=(B,),
        in_specs=[
            pl.BlockSpec((1, N, DIM), lambda b: (b, 0, 0)),
            pl.BlockSpec((1, 1, N), lambda b: (b, 0, 0)),
            pl.BlockSpec((1, N, DIM), lambda b: (b, 0, 0)),
            pl.BlockSpec((3 * DIM, DIM), lambda b: (0, 0)),
            pl.BlockSpec((DIM, DIM), lambda b: (0, 0)),
            pl.BlockSpec((1, DIM), lambda b: (0, 0)),
        ],
        out_specs=pl.BlockSpec((1, N, DIM), lambda b: (b, 0, 0)),
        out_shape=jax.ShapeDtypeStruct((B, N, DIM), _F32),
        compiler_params=pltpu.CompilerParams(
            dimension_semantics=("arbitrary",),
        ),
    )(xr, probs.reshape(B, 1, N), metric, W_qkv, W_proj, b_proj.reshape(1, DIM))
    return out.reshape(B, HGRID, HGRID, DIM)


# speedup vs baseline: 1.6634x; 1.6634x over previous
import jax
import jax.numpy as jnp
from jax import lax
from jax.experimental import pallas as pl
from jax.experimental.pallas import tpu as pltpu

DIM = 768
NUM_HEADS = 12
HEAD_DIM = 64
HGRID = 24
N = HGRID * HGRID
KEEP = int(N * 0.7)
MM = int(0.7 * KEEP)
SCALE = HEAD_DIM ** -0.5
NSRC = (KEEP + 1) // 2
NDST = KEEP // 2
RR = min(KEEP - MM, KEEP // 2)
NUNM = NSRC - RR
TOPK = MM // 8
B = 32

_F32 = jnp.float32
_BF16 = jnp.bfloat16
_HIGH = lax.Precision.HIGHEST


def _bdot(a, b, dims):
    return lax.dot_general(a.astype(_BF16), b.astype(_BF16), (dims, ((), ())),
                           preferred_element_type=_F32)


def _fdot(a, b, dims):
    return lax.dot_general(a, b, (dims, ((), ())),
                           preferred_element_type=_F32, precision=_HIGH)


def _iota(shape, dim):
    return lax.broadcasted_iota(jnp.int32, shape, dim).astype(_F32)


def _body(x_ref, probs_ref, metric_ref, wqkv_ref, wproj_ref, bproj_ref, o_ref):
    xt = x_ref[0]

    i576 = (_iota((N, N), 0) == _iota((N, N), 1)).astype(_F32)
    i202 = (_iota((NSRC, NSRC), 0) == _iota((NSRC, NSRC), 1)).astype(_F32)

    probs_row = probs_ref[0]
    probs_col = _fdot(i576, probs_row, ((1,), (1,)))

    pr_m = jnp.broadcast_to(probs_row, (N, N))
    pr_n = jnp.broadcast_to(probs_col, (N, N))
    lane = _iota((N, N), 1)
    subl = _iota((N, N), 0)
    cmp = (pr_m > pr_n) | ((pr_m == pr_n) & (lane < subl))
    rank_col = jnp.sum(cmp.astype(_F32), axis=1, keepdims=True)
    rank_row = _fdot(rank_col, i576, ((0,), (0,)))

    rr_a = jnp.broadcast_to(rank_row, (NSRC, N))
    p_a = (rr_a == 2.0 * _iota((NSRC, N), 0)).astype(_F32)
    rr_b = jnp.broadcast_to(rank_row, (NDST, N))
    p_b = (rr_b == 2.0 * _iota((NDST, N), 0) + 1.0).astype(_F32)

    metric = metric_ref[0]
    src = _fdot(p_a, xt, ((1,), (0,)))
    dst = _fdot(p_b, xt, ((1,), (0,)))
    a_m = _bdot(p_a, metric, ((1,), (0,)))
    b_m = _bdot(p_b, metric, ((1,), (0,)))

    scores = _bdot(a_m, b_m, ((1,), (1,)))
    msim_col = jnp.max(scores, axis=1, keepdims=True)
    jlane = _iota((NSRC, NDST), 1)
    dstf_col = jnp.min(jnp.where(scores == msim_col, jlane, _F32(NDST)),
                       axis=1, keepdims=True)
    msim_row = _fdot(msim_col, i202, ((0,), (0,)))
    ms_m = jnp.broadcast_to(msim_row, (NSRC, NSRC))
    ms_n = jnp.broadcast_to(msim_col, (NSRC, NSRC))
    lane2 = _iota((NSRC, NSRC), 1)
    subl2 = _iota((NSRC, NSRC), 0)
    cmp2 = (ms_m > ms_n) | ((ms_m == ms_n) & (lane2 < subl2))
    rank2_col = jnp.sum(cmp2.astype(_F32), axis=1, keepdims=True)
    merged_f = (rank2_col < _F32(RR)).astype(_F32)

    s_oh = merged_f * (dstf_col == jlane).astype(_F32)
    sums = _fdot(s_oh, src, ((0,), (0,)))
    counts_col = _fdot(s_oh, jnp.ones((NSRC, 1), _F32), ((0,), (0,)))
    dstm = (dst + sums) / (1.0 + counts_col)

    rank2_row = _fdot(rank2_col, i202, ((0,), (0,)))
    u_oh = (jnp.broadcast_to(rank2_row, (NUNM, NSRC))
            == _F32(RR) + _iota((NUNM, NSRC), 0)).astype(_F32)
    srcu = _bdot(u_oh, src, ((1,), (0,)))

    c1 = (_iota((MM, NUNM), 0) == _iota((MM, NUNM), 1)).astype(_F32)
    c2 = (_iota((MM, NDST), 0) == _iota((MM, NDST), 1) + _F32(NUNM)).astype(_F32)
    xm = _bdot(c1, srcu, ((1,), (0,))) + _fdot(c2, dstm, ((1,), (0,)))

    qkv = _bdot(xm, wqkv_ref[...], ((1,), (1,)))

    outs = []
    for h in range(NUM_HEADS):
        q = qkv[:, h * HEAD_DIM:(h + 1) * HEAD_DIM]
        kk = qkv[:, DIM + h * HEAD_DIM:DIM + (h + 1) * HEAD_DIM]
        v = qkv[:, 2 * DIM + h * HEAD_DIM:2 * DIM + (h + 1) * HEAD_DIM]
        att = _bdot(q, kk, ((1,), (1,))) * _F32(SCALE)
        rmx = jnp.max(att, axis=1, keepdims=True)
        rmn = jnp.min(att, axis=1, keepdims=True)

        def bis_body(_, lh):
            lo, hi = lh
            mid = 0.5 * (lo + hi)
            cnt = jnp.sum(jnp.where(att >= mid, 1.0, 0.0), axis=1, keepdims=True)
            geq = cnt >= _F32(TOPK)
            return jnp.where(geq, mid, lo), jnp.where(geq, hi, mid)

        lo, _ = lax.fori_loop(0, 19, bis_body, (rmn, rmx))
        maskf = jnp.where(att >= lo, 1.0, 0.0)
        e = jnp.exp(att - rmx) * maskf
        p = e / jnp.sum(e, axis=1, keepdims=True)
        outs.append(_bdot(p, v, ((1,), (0,))))

    jl282 = _iota((NSRC, MM), 1)
    dstf_b = jnp.broadcast_to(dstf_col, (NSRC, MM))
    rank2_b = jnp.broadcast_to(rank2_col, (NSRC, MM))
    merged_b = jnp.broadcast_to(merged_f, (NSRC, MM))
    e_oh = (merged_b * (jl282 == _F32(NUNM) + dstf_b).astype(_F32)
            + (1.0 - merged_b) * (jl282 == rank2_b - _F32(RR)).astype(_F32))
    x2 = (_iota((KEEP, NSRC), 0) == 2.0 * _iota((KEEP, NSRC), 1)).astype(_F32)
    pp = _iota((KEEP, MM), 0)
    jj = _iota((KEEP, MM), 1)
    podd = (pp - 2.0 * jnp.floor(pp * 0.5)) == 1.0
    m_odd = (podd & (2.0 * jj == pp + _F32(160.0) + 1.0)).astype(_F32)
    mx_oh = _bdot(x2, e_oh, ((1,), (0,))) + m_odd
    q_oh = (jnp.broadcast_to(rank_col, (N, KEEP))
            == _iota((N, KEEP), 1)).astype(_F32)
    g_oh = _bdot(q_oh, mx_oh, ((1,), (0,)))

    xp = xt * (rank_col >= _F32(KEEP)).astype(_F32)
    xo = _bdot(xp, wproj_ref[...], ((1,), (1,)))
    for h in range(NUM_HEADS):
        yh = _bdot(g_oh, outs[h], ((1,), (0,)))
        wslice = wproj_ref[:, h * HEAD_DIM:(h + 1) * HEAD_DIM]
        xo = xo + _bdot(yh, wslice, ((1,), (1,)))
    xo = xo + bproj_ref[...]
    o_ref[0] = xo


@jax.jit
def kernel(x, W_g, W_k, W_qkv, W_proj, b_proj):
    xr = x.reshape(B, N, DIM)
    gt = (xr.mean(axis=1) @ W_g.T)[:, None, :]
    k = xr @ W_k.T
    gs = jnp.einsum('bqd,bnd->bqn', gt, k) * SCALE
    probs = jax.nn.softmax(gs[:, 0, :], axis=-1)
    metric = k / jnp.linalg.norm(k, axis=-1, keepdims=True)
    out = pl.pallas_call(
        _body,
        grid=(B,),
        in_specs=[
            pl.BlockSpec((1, N, DIM), lambda b: (b, 0, 0)),
            pl.BlockSpec((1, 1, N), lambda b: (b, 0, 0)),
            pl.BlockSpec((1, N, DIM), lambda b: (b, 0, 0)),
            pl.BlockSpec((3 * DIM, DIM), lambda b: (0, 0)),
            pl.BlockSpec((DIM, DIM), lambda b: (0, 0)),
            pl.BlockSpec((1, DIM), lambda b: (0, 0)),
        ],
        out_specs=pl.BlockSpec((1, N, DIM), lambda b: (b, 0, 0)),
        out_shape=jax.ShapeDtypeStruct((B, N, DIM), _F32),
        compiler_params=pltpu.CompilerParams(
            dimension_semantics=("arbitrary",),
        ),
    )(xr, probs.reshape(B, 1, N), metric, W_qkv, W_proj, b_proj.reshape(1, DIM))
    return out.reshape(B, HGRID, HGRID, DIM)
